# Initial kernel scaffold; baseline (speedup 1.0000x reference)
import functools

import jax
import jax.numpy as jnp
from jax import lax
from jax.experimental import pallas as pl
from jax.experimental.pallas import tpu as pltpu
from jax.experimental.pallas import tpu_sc as plsc

N = 10000
E = 320000

NC = 2
NS = 16
L = 16

EDGES_PER_TILE = E // (NC * NS)
CHUNK = 80
NCHUNK = EDGES_PER_TILE // CHUNK
ROWS_PER_TILE = N // NS


def _spmm_body(F, src_hbm, dst_hbm, ew_hbm, sup_hbm, out_hbm,
               srcb, dstb, ewb, rows, zrow, accum, sem):
  c = lax.axis_index("c")
  s = lax.axis_index("s")
  base = c * (E // NC) + s * EDGES_PER_TILE

  pltpu.sync_copy(src_hbm.at[pl.ds(base, EDGES_PER_TILE)], srcb)
  pltpu.sync_copy(dst_hbm.at[c, s], dstb)
  pltpu.sync_copy(ew_hbm.at[pl.ds(base, EDGES_PER_TILE)], ewb)

  for q in range(F // L):
    zrow[pl.ds(q * L, L)] = jnp.zeros((L,), jnp.float32)

  def zero_row(r, _):
    pltpu.sync_copy(zrow, accum.at[s * ROWS_PER_TILE + r])
    return 0
  lax.fori_loop(0, ROWS_PER_TILE, zero_row, 0)
  plsc.subcore_barrier()

  def chunk_body(j, _):
    pltpu.make_async_copy(
        sup_hbm.at[srcb.at[pl.ds(j * CHUNK, CHUNK)]], rows, sem).start()
    pltpu.make_async_copy(
        sup_hbm.at[srcb.at[pl.ds(j * CHUNK, CHUNK)]], rows, sem).wait()

    def scale_edge(e, _):
      g = j * CHUNK + (e // L) * L
      ew_v = ewb[pl.ds(g, L)]
      lane = jnp.full((L,), e % L, jnp.int32)
      spl = jnp.take(ew_v, lane)
      for q in range(F // L):
        rows[e, pl.ds(q * L, L)] = rows[e, pl.ds(q * L, L)] * spl
      return 0
    lax.fori_loop(0, CHUNK, scale_edge, 0)

    pltpu.sync_copy(rows, accum.at[dstb.at[j]], add=True)
    return 0

  lax.fori_loop(0, NCHUNK, chunk_body, 0)
  plsc.subcore_barrier()

  pltpu.sync_copy(accum.at[pl.ds(s * ROWS_PER_TILE, ROWS_PER_TILE)],
                  out_hbm.at[c, pl.ds(s * ROWS_PER_TILE, ROWS_PER_TILE)])


def _make_spmm(F):
  mesh = plsc.VectorSubcoreMesh(core_axis_name="c", subcore_axis_name="s")
  return pl.kernel(
      functools.partial(_spmm_body, F),
      out_type=jax.ShapeDtypeStruct((NC, N, F), jnp.float32),
      mesh=mesh,
      scratch_types=[
          pltpu.VMEM((EDGES_PER_TILE,), jnp.int32),
          pltpu.VMEM((NCHUNK, CHUNK), jnp.int32),
          pltpu.VMEM((EDGES_PER_TILE,), jnp.float32),
          pltpu.VMEM((CHUNK, F), jnp.float32),
          pltpu.VMEM((F,), jnp.float32),
          pltpu.VMEM_SHARED((N, F), jnp.float32),
          pltpu.SemaphoreType.DMA,
      ],
      name=f"spmm_sc_f{F}",
  )


_spmm_128 = _make_spmm(128)
_spmm_64 = _make_spmm(64)
_spmm_48 = _make_spmm(48)


def _spmm(sup, src, dst2d, ew, F):
  if F == 128:
    return _spmm_128(src, dst2d, ew, sup)
  if F == 64:
    return _spmm_64(src, dst2d, ew, sup)
  return _spmm_48(src, dst2d, ew, sup)


BLK = 1000


def _mm_body(x_ref, w_ref, o_ref):
  o_ref[...] = jnp.dot(x_ref[...], w_ref[...],
                       preferred_element_type=jnp.float32)


def _mm(x, w):
  n, k = x.shape
  m = w.shape[1]
  return pl.pallas_call(
      _mm_body,
      grid=(n // BLK,),
      in_specs=[pl.BlockSpec((BLK, k), lambda i: (i, 0)),
                pl.BlockSpec((k, m), lambda i: (0, 0))],
      out_specs=pl.BlockSpec((BLK, m), lambda i: (i, 0)),
      out_shape=jax.ShapeDtypeStruct((n, m), jnp.float32),
  )(x, w)


def _fuse1_body(p_ref, b1_ref, w2_ref, ew_ref, eb_ref, s2_ref, o2_ref):
  h = jax.nn.relu(p_ref[0] + p_ref[1] + b1_ref[...])
  s2_ref[...] = jnp.dot(h, w2_ref[...], preferred_element_type=jnp.float32)
  o2_ref[...] = jnp.dot(h, ew_ref[...],
                        preferred_element_type=jnp.float32) + eb_ref[...]


def _fuse1(p, b1, w2, encw, encb):
  return pl.pallas_call(
      _fuse1_body,
      grid=(N // BLK,),
      in_specs=[pl.BlockSpec((2, BLK, 128), lambda i: (0, i, 0)),
                pl.BlockSpec((1, 128), lambda i: (0, 0)),
                pl.BlockSpec((128, 64), lambda i: (0, 0)),
                pl.BlockSpec((128, 16), lambda i: (0, 0)),
                pl.BlockSpec((1, 16), lambda i: (0, 0))],
      out_specs=[pl.BlockSpec((BLK, 64), lambda i: (i, 0)),
                 pl.BlockSpec((BLK, 16), lambda i: (i, 0))],
      out_shape=[jax.ShapeDtypeStruct((N, 64), jnp.float32),
                 jax.ShapeDtypeStruct((N, 16), jnp.float32)],
  )(p, b1, w2, encw, encb)


def _fuse2_body(p_ref, b2_ref, w3_ref, s3_ref):
  h = jax.nn.relu(p_ref[0] + p_ref[1] + b2_ref[...])
  s3_ref[...] = jnp.dot(h, w3_ref[...], preferred_element_type=jnp.float32)


def _fuse2(p, b2, w3p):
  return pl.pallas_call(
      _fuse2_body,
      grid=(N // BLK,),
      in_specs=[pl.BlockSpec((2, BLK, 64), lambda i: (0, i, 0)),
                pl.BlockSpec((1, 64), lambda i: (0, 0)),
                pl.BlockSpec((64, 48), lambda i: (0, 0))],
      out_specs=pl.BlockSpec((BLK, 48), lambda i: (i, 0)),
      out_shape=jax.ShapeDtypeStruct((N, 48), jnp.float32),
  )(p, b2, w3p)


def _final_body(p_ref, b3_ref, o_ref):
  z = p_ref[0] + p_ref[1] + b3_ref[...]
  m = jnp.max(z, axis=1, keepdims=True)
  lse = jnp.log(jnp.sum(jnp.exp(z - m), axis=1, keepdims=True))
  o_ref[...] = z - m - lse


def _final(p, b3p):
  return pl.pallas_call(
      _final_body,
      grid=(N // BLK,),
      in_specs=[pl.BlockSpec((2, BLK, 48), lambda i: (0, i, 0)),
                pl.BlockSpec((1, 48), lambda i: (0, 0))],
      out_specs=pl.BlockSpec((BLK, 48), lambda i: (i, 0)),
      out_shape=jax.ShapeDtypeStruct((N, 48), jnp.float32),
  )(p, b3p)


@jax.jit
def kernel(x, edge_index, edge_weight, W1, b1, W2, b2, W3, b3, encW, encb):
  src = edge_index[0]
  dst = edge_index[1]
  dst2d = dst.reshape(NC, NS, NCHUNK, CHUNK)

  w3p = jnp.pad(W3, ((0, 0), (0, 8)))
  b3p = jnp.concatenate([b3, jnp.full((8,), -1e30, jnp.float32)])

  sup1 = _mm(x, W1)
  p1 = _spmm(sup1, src, dst2d, edge_weight, 128)
  sup2, out2 = _fuse1(p1, b1.reshape(1, -1), W2, encW, encb.reshape(1, -1))
  p2 = _spmm(sup2, src, dst2d, edge_weight, 64)
  sup3 = _fuse2(p2, b2.reshape(1, -1), w3p)
  p3 = _spmm(sup3, src, dst2d, edge_weight, 48)
  out1 = _final(p3, b3p.reshape(1, -1))
  return (out1[:, :40], out2)


# speedup vs baseline: 5.1271x; 5.1271x over previous
import functools

import jax
import jax.numpy as jnp
from jax import lax
from jax.experimental import pallas as pl
from jax.experimental.pallas import tpu as pltpu
from jax.experimental.pallas import tpu_sc as plsc

N = 10000
NP = 10240
E = 320000

NC = 2
NS = 16
L = 16

EDGES_PER_TILE = E // (NC * NS)
CHUNK = 80
NCHUNK = EDGES_PER_TILE // CHUNK
ROWS_PER_TILE = NP // NS
ZROWS = 16


def _spmm_body(F, src_hbm, dst_hbm, ew_hbm, sup_hbm, out_hbm,
               srcb, dstb, ewb, rows, zbuf, accum, sem):
  c = lax.axis_index("c")
  s = lax.axis_index("s")
  base = c * (E // NC) + s * EDGES_PER_TILE

  pltpu.sync_copy(src_hbm.at[pl.ds(base, EDGES_PER_TILE)], srcb)
  pltpu.sync_copy(ew_hbm.at[pl.ds(base, EDGES_PER_TILE)], ewb)

  for r in range(ZROWS):
    for q in range(F // L):
      zbuf[r, pl.ds(q * L, L)] = jnp.zeros((L,), jnp.float32)

  def zero_rows(k, _):
    pltpu.sync_copy(zbuf, accum.at[pl.ds(s * ROWS_PER_TILE + k * ZROWS,
                                         ZROWS)])
    return 0
  lax.fori_loop(0, ROWS_PER_TILE // ZROWS, zero_rows, 0)
  plsc.subcore_barrier()

  def chunk_body(j, _):
    pltpu.sync_copy(dst_hbm.at[c, s, j], dstb)
    pltpu.make_async_copy(
        sup_hbm.at[srcb.at[pl.ds(j * CHUNK, CHUNK)]], rows, sem).start()
    pltpu.make_async_copy(
        sup_hbm.at[srcb.at[pl.ds(j * CHUNK, CHUNK)]], rows, sem).wait()

    def scale_edge(e, _):
      g = j * CHUNK + (e // L) * L
      ew_v = ewb[pl.ds(g, L)]
      lane = jnp.full((L,), e % L, jnp.int32)
      spl = lax.gather(
          ew_v, lane[:, None],
          lax.GatherDimensionNumbers(offset_dims=(), collapsed_slice_dims=(0,),
                                     start_index_map=(0,)),
          (1,), mode=lax.GatherScatterMode.PROMISE_IN_BOUNDS)
      for q in range(F // L):
        rows[e, pl.ds(q * L, L)] = rows[e, pl.ds(q * L, L)] * spl
      return 0
    lax.fori_loop(0, CHUNK, scale_edge, 0)

    pltpu.sync_copy(rows, accum.at[dstb], add=True)
    return 0

  lax.fori_loop(0, NCHUNK, chunk_body, 0)
  plsc.subcore_barrier()

  pltpu.sync_copy(accum.at[pl.ds(s * ROWS_PER_TILE, ROWS_PER_TILE)],
                  out_hbm.at[c, pl.ds(s * ROWS_PER_TILE, ROWS_PER_TILE)])


def _make_spmm(F):
  mesh = plsc.VectorSubcoreMesh(core_axis_name="c", subcore_axis_name="s")
  return pl.kernel(
      functools.partial(_spmm_body, F),
      out_type=jax.ShapeDtypeStruct((NC, NP, F), jnp.float32),
      mesh=mesh,
      scratch_types=[
          pltpu.VMEM((EDGES_PER_TILE,), jnp.int32),
          pltpu.VMEM((CHUNK,), jnp.int32),
          pltpu.VMEM((EDGES_PER_TILE,), jnp.float32),
          pltpu.VMEM((CHUNK, F), jnp.float32),
          pltpu.VMEM((ZROWS, F), jnp.float32),
          pltpu.VMEM_SHARED((NP, F), jnp.float32),
          pltpu.SemaphoreType.DMA,
      ],
      compiler_params=pltpu.CompilerParams(use_tc_tiling_on_sc=False),
      name=f"spmm_sc_f{F}",
  )


_spmm_128 = _make_spmm(128)
_spmm_64 = _make_spmm(64)
_spmm_48 = _make_spmm(48)


def _spmm(sup, src, dst2d, ew, F):
  if F == 128:
    return _spmm_128(src, dst2d, ew, sup)
  if F == 64:
    return _spmm_64(src, dst2d, ew, sup)
  return _spmm_48(src, dst2d, ew, sup)


BLK = 1024


def _mm_body(x_ref, w_ref, o_ref):
  o_ref[...] = jnp.dot(x_ref[...], w_ref[...],
                       preferred_element_type=jnp.float32)


def _mm(x, w):
  n, k = x.shape
  m = w.shape[1]
  return pl.pallas_call(
      _mm_body,
      grid=(n // BLK,),
      in_specs=[pl.BlockSpec((BLK, k), lambda i: (i, 0)),
                pl.BlockSpec((k, m), lambda i: (0, 0))],
      out_specs=pl.BlockSpec((BLK, m), lambda i: (i, 0)),
      out_shape=jax.ShapeDtypeStruct((n, m), jnp.float32),
  )(x, w)


def _fuse1_body(p_ref, b1_ref, w2_ref, ew_ref, eb_ref, s2_ref, o2_ref):
  h = jax.nn.relu(p_ref[0] + p_ref[1] + b1_ref[...])
  s2_ref[...] = jnp.dot(h, w2_ref[...], preferred_element_type=jnp.float32)
  o2_ref[...] = jnp.dot(h, ew_ref[...],
                        preferred_element_type=jnp.float32) + eb_ref[...]


def _fuse1(p, b1, w2, encw, encb):
  return pl.pallas_call(
      _fuse1_body,
      grid=(NP // BLK,),
      in_specs=[pl.BlockSpec((2, BLK, 128), lambda i: (0, i, 0)),
                pl.BlockSpec((1, 128), lambda i: (0, 0)),
                pl.BlockSpec((128, 64), lambda i: (0, 0)),
                pl.BlockSpec((128, 16), lambda i: (0, 0)),
                pl.BlockSpec((1, 16), lambda i: (0, 0))],
      out_specs=[pl.BlockSpec((BLK, 64), lambda i: (i, 0)),
                 pl.BlockSpec((BLK, 16), lambda i: (i, 0))],
      out_shape=[jax.ShapeDtypeStruct((NP, 64), jnp.float32),
                 jax.ShapeDtypeStruct((NP, 16), jnp.float32)],
  )(p, b1, w2, encw, encb)


def _fuse2_body(p_ref, b2_ref, w3_ref, s3_ref):
  h = jax.nn.relu(p_ref[0] + p_ref[1] + b2_ref[...])
  s3_ref[...] = jnp.dot(h, w3_ref[...], preferred_element_type=jnp.float32)


def _fuse2(p, b2, w3p):
  return pl.pallas_call(
      _fuse2_body,
      grid=(NP // BLK,),
      in_specs=[pl.BlockSpec((2, BLK, 64), lambda i: (0, i, 0)),
                pl.BlockSpec((1, 64), lambda i: (0, 0)),
                pl.BlockSpec((64, 48), lambda i: (0, 0))],
      out_specs=pl.BlockSpec((BLK, 48), lambda i: (i, 0)),
      out_shape=jax.ShapeDtypeStruct((NP, 48), jnp.float32),
  )(p, b2, w3p)


def _final_body(p_ref, b3_ref, o_ref):
  z = p_ref[0] + p_ref[1] + b3_ref[...]
  m = jnp.max(z, axis=1, keepdims=True)
  lse = jnp.log(jnp.sum(jnp.exp(z - m), axis=1, keepdims=True))
  o_ref[...] = z - m - lse


def _final(p, b3p):
  return pl.pallas_call(
      _final_body,
      grid=(NP // BLK,),
      in_specs=[pl.BlockSpec((2, BLK, 48), lambda i: (0, i, 0)),
                pl.BlockSpec((1, 48), lambda i: (0, 0))],
      out_specs=pl.BlockSpec((BLK, 48), lambda i: (i, 0)),
      out_shape=jax.ShapeDtypeStruct((NP, 48), jnp.float32),
  )(p, b3p)


@jax.jit
def kernel(x, edge_index, edge_weight, W1, b1, W2, b2, W3, b3, encW, encb):
  src = edge_index[0]
  dst = edge_index[1]
  dst2d = dst.reshape(NC, NS, NCHUNK, CHUNK)

  xp = jnp.pad(x, ((0, NP - N), (0, 0)))
  w3p = jnp.pad(W3, ((0, 0), (0, 8)))
  b3p = jnp.concatenate([b3, jnp.full((8,), -1e30, jnp.float32)])

  sup1 = _mm(xp, W1)
  p1 = _spmm(sup1, src, dst2d, edge_weight, 128)
  sup2, out2 = _fuse1(p1, b1.reshape(1, -1), W2, encW, encb.reshape(1, -1))
  p2 = _spmm(sup2, src, dst2d, edge_weight, 64)
  sup3 = _fuse2(p2, b2.reshape(1, -1), w3p)
  p3 = _spmm(sup3, src, dst2d, edge_weight, 48)
  out1 = _final(p3, b3p.reshape(1, -1))
  return (out1[:N, :40], out2[:N])
